# baseline (speedup 1.0000x reference)
import functools

import jax
import jax.numpy as jnp
from jax import lax
from jax.experimental import pallas as pl
from jax.experimental.pallas import tpu as pltpu
from jax.experimental.pallas import tpu_sc as plsc

NC = 2
NS = 16
NW = NC * NS
CH = 80
PACK = 16384


def _sc_mesh():
    return plsc.VectorSubcoreMesh(
        core_axis_name="c", subcore_axis_name="s", num_cores=NC, num_subcores=NS
    )


def _make_deg_kernel(E, N):
    PER_W = E // NW
    K = PER_W // CH
    assert K * CH == PER_W and PER_W * NW == E
    zb_rows = N // CH
    assert zb_rows * CH == N and zb_rows % 5 == 0
    assert PER_W % 16 == 0
    CHW = ((PER_W + 112 + 127) // 128) * 128
    last_a = ((NW - 1) * PER_W // 128) * 128
    assert last_a + CHW <= E

    @functools.partial(
        pl.kernel,
        mesh=_sc_mesh(),
        out_type=(jax.ShapeDtypeStruct((NC, N), jnp.float32),
                  jax.ShapeDtypeStruct((E,), jnp.int32)),
        scratch_types=[
            pltpu.VMEM((2, CHW), jnp.int32),
            pltpu.VMEM((PER_W,), jnp.int32),
            pltpu.VMEM((K, CH), jnp.int32),
            pltpu.VMEM((CH,), jnp.float32),
            pltpu.VMEM((CH,), jnp.float32),
            pltpu.SemaphoreType.DMA,
            pltpu.SemaphoreType.DMA,
            pltpu.VMEM_SHARED((N,), jnp.float32),
        ],
    )
    def deg_kernel(ei_hbm, deg_out, packed_out,
                   ei_v, packed_b, dst2d, ones_v, zero_v, isem, psem, deg_sh):
        c = lax.axis_index("c")
        s = lax.axis_index("s")
        wid = s * NC + c

        base = wid * PER_W
        a = pl.multiple_of((base // 128) * 128, 128)
        local = base - a
        d_ei = pltpu.async_copy(ei_hbm.at[:, pl.ds(a, CHW)], ei_v, isem)
        for i in range(CH // 16):
            ones_v[pl.ds(i * 16, 16)] = jnp.ones((16,), jnp.float32)
            zero_v[pl.ds(i * 16, 16)] = jnp.zeros((16,), jnp.float32)

        @pl.when(s < 5)
        def _():
            for j in range(zb_rows // 5):
                pltpu.sync_copy(
                    zero_v, deg_sh.at[pl.ds((s * (zb_rows // 5) + j) * CH,
                                            CH)])
        d_ei.wait()

        def unp(k, carry):
            for i in range(CH // 16):
                o = pl.ds(local + k * CH + i * 16, 16)
                vd = ei_v[1, o]
                dst2d[k, pl.ds(i * 16, 16)] = vd
                packed_b[pl.ds(k * CH + i * 16, 16)] = ei_v[0, o] * PACK + vd
            return carry

        lax.fori_loop(0, K, unp, 0)
        d_pack = pltpu.async_copy(packed_b,
                                  packed_out.at[pl.ds(base, PER_W)], psem)
        plsc.subcore_barrier()

        descs = [
            pltpu.async_copy(ones_v, deg_sh.at[dst2d.at[k]], isem, add=True)
            for k in range(K)
        ]
        for d in descs:
            d.wait()
        d_pack.wait()
        plsc.subcore_barrier()

        @pl.when(s == 0)
        def _():
            pltpu.sync_copy(deg_sh, deg_out.at[c])

    return deg_kernel


def _make_scatter_kernel(E, N, D):
    PER_W = E // NW
    K = PER_W // CH
    assert K * CH == PER_W and PER_W * NW == E
    n_str = N // CH
    assert n_str * CH == N and CH % 8 == 0

    NB = 3
    HA = (K + 1) // 2
    halves = [(0, HA), (HA, K - HA)]

    @functools.partial(
        pl.kernel,
        mesh=_sc_mesh(),
        out_type=jax.ShapeDtypeStruct((NC, N, D), jnp.float32),
        scratch_types=[
            pltpu.VMEM((HA * CH,), jnp.int32),
            [pltpu.VMEM((CH,), jnp.int32) for _ in range(NB)],
            [pltpu.VMEM((CH,), jnp.int32) for _ in range(NB)],
            [pltpu.VMEM((CH, D), jnp.float32) for _ in range(NB)],
            [pltpu.SemaphoreType.DMA for _ in range(NB)],
            [pltpu.SemaphoreType.DMA for _ in range(NB)],
            pltpu.VMEM_SHARED((N, D), jnp.float32),
        ],
    )
    def scatter_kernel(g_hbm, packed_hbm, out_hbm,
                       packed_v, srcb, dstb, rows, gsems, ssems, acc_sh):
        c = lax.axis_index("c")
        s = lax.axis_index("s")
        wid = s * NC + c

        sq, sr = divmod(n_str, NS)

        def each_stripe(fn):
            @pl.when(s < sr)
            def _():
                for j in range(sq + 1):
                    fn(s * (sq + 1) + j)

            @pl.when(s >= sr)
            def _():
                for j in range(sq):
                    fn(sr * (sq + 1) + (s - sr) * sq + j)

        def zrow(r, carry):
            for i in range(D // 16):
                rows[0][r, pl.ds(i * 16, 16)] = jnp.zeros((16,), jnp.float32)
            return carry

        lax.fori_loop(0, CH, zrow, 0)
        each_stripe(lambda t: pltpu.sync_copy(
            rows[0], acc_sh.at[pl.ds(t * CH, CH)]))

        plsc.subcore_barrier()

        def unpack(j, b):
            for i in range(CH // 16):
                v = packed_v[pl.ds(j * CH + i * 16, 16)]
                srcb[b][pl.ds(i * 16, 16)] = lax.shift_right_logical(v, 14)
                dstb[b][pl.ds(i * 16, 16)] = lax.bitwise_and(v, PACK - 1)

        def start_gather(b):
            pltpu.async_copy(g_hbm.at[srcb[b]], rows[b], gsems[b])

        def wait_gather(b):
            pltpu.make_async_copy(g_hbm.at[srcb[b]], rows[b], gsems[b]).wait()

        def start_scatter(b):
            pltpu.async_copy(rows[b], acc_sh.at[dstb[b]], ssems[b], add=True)

        def wait_scatter(b):
            pltpu.make_async_copy(rows[b], acc_sh.at[dstb[b]],
                                  ssems[b]).wait()

        for base, nch in halves:
            pltpu.sync_copy(
                packed_hbm.at[pl.ds(wid * PER_W + base * CH, nch * CH)],
                packed_v.at[pl.ds(0, nch * CH)])
            for b in range(NB):
                unpack(b, b)
                start_gather(b)
            NG = nch // NB

            def body(g, carry, nch=nch):
                for b in range(NB):
                    j = g * NB + b
                    wait_gather(b)
                    start_scatter(b)

                    @pl.when(j + NB < nch)
                    def _(j=j, b=b):
                        wait_scatter(b)
                        unpack(j + NB, b)
                        start_gather(b)

                return carry

            lax.fori_loop(0, NG, body, 0)
            for j in range(NG * NB, nch):
                wait_gather(j % NB)
                pltpu.sync_copy(rows[j % NB], acc_sh.at[dstb[j % NB]],
                                add=True)
            for j in range(max(nch - NB, 0), NG * NB):
                wait_scatter(j % NB)

        plsc.subcore_barrier()

        each_stripe(lambda t: pltpu.sync_copy(
            acc_sh.at[pl.ds(t * CH, CH)],
            out_hbm.at[c, pl.ds(t * CH, CH)]))

    return scatter_kernel


def _mm_body(x_ref, w_ref, dis_ref, g_ref):
    h = lax.dot_general(x_ref[...], w_ref[...], (((1,), (1,)), ((), ())),
                        preferred_element_type=jnp.float32)
    g_ref[...] = h * dis_ref[...]


def _epilogue_body(acc_ref, g_ref, dis_ref, b_ref, out_ref):
    total = acc_ref[0] + acc_ref[1] + g_ref[...]
    out_ref[...] = jnp.maximum(total * dis_ref[...] + b_ref[...], 0.0)


def kernel(x, edge_index, edge_attr, batch, y, W, b):
    N, D = x.shape
    E = edge_index.shape[1]
    assert N <= PACK

    pdeg, packed = _make_deg_kernel(E, N)(edge_index)
    deg = pdeg[0] + pdeg[1] + 1.0
    dis = lax.rsqrt(deg)[:, None]

    BN = 2000
    g = pl.pallas_call(
        _mm_body,
        grid=(N // BN,),
        in_specs=[
            pl.BlockSpec((BN, D), lambda i: (i, 0)),
            pl.BlockSpec((D, D), lambda i: (0, 0)),
            pl.BlockSpec((BN, 1), lambda i: (i, 0)),
        ],
        out_specs=pl.BlockSpec((BN, D), lambda i: (i, 0)),
        out_shape=jax.ShapeDtypeStruct((N, D), jnp.float32),
    )(x, W, dis)

    acc = _make_scatter_kernel(E, N, D)(g, packed)

    out = pl.pallas_call(
        _epilogue_body,
        grid=(N // BN,),
        in_specs=[
            pl.BlockSpec((NC, BN, D), lambda i: (0, i, 0)),
            pl.BlockSpec((BN, D), lambda i: (i, 0)),
            pl.BlockSpec((BN, 1), lambda i: (i, 0)),
            pl.BlockSpec((1, D), lambda i: (0, 0)),
        ],
        out_specs=pl.BlockSpec((BN, D), lambda i: (i, 0)),
        out_shape=jax.ShapeDtypeStruct((N, D), jnp.float32),
    )(acc, g, dis, b[None, :])
    return out


# speedup vs baseline: 1.0073x; 1.0073x over previous
import functools

import jax
import jax.numpy as jnp
from jax import lax
from jax.experimental import pallas as pl
from jax.experimental.pallas import tpu as pltpu
from jax.experimental.pallas import tpu_sc as plsc

NC = 2
NS = 16
NW = NC * NS
CH = 80
PACK = 16384


def _sc_mesh():
    return plsc.VectorSubcoreMesh(
        core_axis_name="c", subcore_axis_name="s", num_cores=NC, num_subcores=NS
    )


def _make_deg_kernel(E, N):
    PER_W = E // NW
    K = PER_W // CH
    assert K * CH == PER_W and PER_W * NW == E
    zb_rows = N // CH
    assert zb_rows * CH == N and zb_rows % 5 == 0
    assert PER_W % 16 == 0
    CHW = ((PER_W + 112 + 127) // 128) * 128
    last_a = ((NW - 1) * PER_W // 128) * 128
    assert last_a + CHW <= E

    @functools.partial(
        pl.kernel,
        mesh=_sc_mesh(),
        out_type=(jax.ShapeDtypeStruct((NC, N), jnp.float32),
                  jax.ShapeDtypeStruct((E,), jnp.int32)),
        scratch_types=[
            pltpu.VMEM((2, CHW), jnp.int32),
            pltpu.VMEM((PER_W,), jnp.int32),
            pltpu.VMEM((K, CH), jnp.int32),
            pltpu.VMEM((CH,), jnp.float32),
            pltpu.VMEM((CH,), jnp.float32),
            pltpu.SemaphoreType.DMA,
            pltpu.SemaphoreType.DMA,
            pltpu.VMEM_SHARED((N,), jnp.float32),
        ],
    )
    def deg_kernel(ei_hbm, deg_out, packed_out,
                   ei_v, packed_b, dst2d, ones_v, zero_v, isem, psem, deg_sh):
        c = lax.axis_index("c")
        s = lax.axis_index("s")
        wid = s * NC + c

        base = wid * PER_W
        a = pl.multiple_of((base // 128) * 128, 128)
        local = base - a
        d_ei = pltpu.async_copy(ei_hbm.at[:, pl.ds(a, CHW)], ei_v, isem)
        for i in range(CH // 16):
            ones_v[pl.ds(i * 16, 16)] = jnp.ones((16,), jnp.float32)
            zero_v[pl.ds(i * 16, 16)] = jnp.zeros((16,), jnp.float32)

        @pl.when(s < 5)
        def _():
            for j in range(zb_rows // 5):
                pltpu.sync_copy(
                    zero_v, deg_sh.at[pl.ds((s * (zb_rows // 5) + j) * CH,
                                            CH)])
        d_ei.wait()

        def unp(k, carry):
            for i in range(CH // 16):
                o = pl.ds(local + k * CH + i * 16, 16)
                vd = ei_v[1, o]
                dst2d[k, pl.ds(i * 16, 16)] = vd
                packed_b[pl.ds(k * CH + i * 16, 16)] = ei_v[0, o] * PACK + vd
            return carry

        lax.fori_loop(0, K, unp, 0)
        d_pack = pltpu.async_copy(packed_b,
                                  packed_out.at[pl.ds(base, PER_W)], psem)
        plsc.subcore_barrier()

        descs = [
            pltpu.async_copy(ones_v, deg_sh.at[dst2d.at[k]], isem, add=True)
            for k in range(K)
        ]
        for d in descs:
            d.wait()
        d_pack.wait()
        plsc.subcore_barrier()

        @pl.when(s == 0)
        def _():
            pltpu.sync_copy(deg_sh, deg_out.at[c])

    return deg_kernel


def _make_scatter_kernel(E, N, D):
    PER_W = E // NW
    K = PER_W // CH
    assert K * CH == PER_W and PER_W * NW == E
    n_str = N // CH
    assert n_str * CH == N and CH % 8 == 0

    NB = 3
    HA = (K + 1) // 2
    halves = [(0, HA), (HA, K - HA)]

    @functools.partial(
        pl.kernel,
        mesh=_sc_mesh(),
        out_type=jax.ShapeDtypeStruct((NC, N, D), jnp.float32),
        scratch_types=[
            pltpu.VMEM((HA * CH,), jnp.int32),
            [pltpu.VMEM((CH,), jnp.int32) for _ in range(NB)],
            [pltpu.VMEM((CH,), jnp.int32) for _ in range(NB)],
            [pltpu.VMEM((CH, D), jnp.float32) for _ in range(NB)],
            [pltpu.SemaphoreType.DMA for _ in range(NB)],
            [pltpu.SemaphoreType.DMA for _ in range(NB)],
            pltpu.VMEM_SHARED((N, D), jnp.float32),
        ],
    )
    def scatter_kernel(g_hbm, packed_hbm, out_hbm,
                       packed_v, srcb, dstb, rows, gsems, ssems, acc_sh):
        c = lax.axis_index("c")
        s = lax.axis_index("s")
        wid = s * NC + c

        sq, sr = divmod(n_str, NS)

        def each_stripe(fn):
            @pl.when(s < sr)
            def _():
                for j in range(sq + 1):
                    fn(s * (sq + 1) + j)

            @pl.when(s >= sr)
            def _():
                for j in range(sq):
                    fn(sr * (sq + 1) + (s - sr) * sq + j)

        def zrow(r, carry):
            for i in range(D // 16):
                rows[0][r, pl.ds(i * 16, 16)] = jnp.zeros((16,), jnp.float32)
            return carry

        lax.fori_loop(0, CH, zrow, 0)
        each_stripe(lambda t: pltpu.sync_copy(
            rows[0], acc_sh.at[pl.ds(t * CH, CH)]))

        plsc.subcore_barrier()

        def unpack(j, b):
            for i in range(CH // 16):
                v = packed_v[pl.ds(j * CH + i * 16, 16)]
                srcb[b][pl.ds(i * 16, 16)] = lax.shift_right_logical(v, 14)
                dstb[b][pl.ds(i * 16, 16)] = lax.bitwise_and(v, PACK - 1)

        def start_gather(b):
            pltpu.async_copy(g_hbm.at[srcb[b]], rows[b], gsems[b])

        def wait_gather(b):
            pltpu.make_async_copy(g_hbm.at[srcb[b]], rows[b], gsems[b]).wait()

        def start_scatter(b):
            pltpu.async_copy(rows[b], acc_sh.at[dstb[b]], ssems[b], add=True)

        def wait_scatter(b):
            pltpu.make_async_copy(rows[b], acc_sh.at[dstb[b]],
                                  ssems[b]).wait()

        for base, nch in halves:
            pltpu.sync_copy(
                packed_hbm.at[pl.ds(wid * PER_W + base * CH, nch * CH)],
                packed_v.at[pl.ds(0, nch * CH)])
            for b in range(NB):
                unpack(b, b)
                start_gather(b)
            NG = nch // NB

            def body(g, carry, nch=nch):
                for b in range(NB):
                    j = g * NB + b
                    wait_gather(b)
                    start_scatter(b)

                    @pl.when(j + NB < nch)
                    def _(j=j, b=b):
                        wait_scatter(b)
                        unpack(j + NB, b)
                        start_gather(b)

                return carry

            lax.fori_loop(0, NG, body, 0)
            for j in range(NG * NB, nch):
                wait_gather(j % NB)
                pltpu.sync_copy(rows[j % NB], acc_sh.at[dstb[j % NB]],
                                add=True)
            for j in range(max(nch - NB, 0), NG * NB):
                wait_scatter(j % NB)

        plsc.subcore_barrier()

        each_stripe(lambda t: pltpu.sync_copy(
            acc_sh.at[pl.ds(t * CH, CH)],
            out_hbm.at[c, pl.ds(t * CH, CH)]))

    return scatter_kernel


def _mm_body(x_ref, w_ref, dis_ref, g_ref):
    h = lax.dot_general(x_ref[...], w_ref[...], (((1,), (1,)), ((), ())),
                        preferred_element_type=jnp.float32)
    g_ref[...] = h * dis_ref[...]


def _epilogue_body(acc_ref, g_ref, dis_ref, b_ref, out_ref):
    total = acc_ref[0] + acc_ref[1] + g_ref[...]
    out_ref[...] = jnp.maximum(total * dis_ref[...] + b_ref[...], 0.0)


def kernel(x, edge_index, edge_attr, batch, y, W, b):
    N, D = x.shape
    E = edge_index.shape[1]
    assert N <= PACK

    pdeg, packed = _make_deg_kernel(E, N)(edge_index)
    deg = pdeg[0] + pdeg[1] + 1.0
    dis = lax.rsqrt(deg)[:, None]

    g = pl.pallas_call(
        _mm_body,
        out_shape=jax.ShapeDtypeStruct((N, D), jnp.float32),
    )(x, W, dis)

    acc = _make_scatter_kernel(E, N, D)(g, packed)

    out = pl.pallas_call(
        _epilogue_body,
        out_shape=jax.ShapeDtypeStruct((N, D), jnp.float32),
    )(acc, g, dis, b[None, :])
    return out
